# Initial kernel scaffold; baseline (speedup 1.0000x reference)
import jax
import jax.numpy as jnp
from jax.experimental import pallas as pl


def kernel(x_friend, x_follow, x_group, x_event, src_friend, src_follow, src_group, src_event, W, b, gamma, beta):
    raise NotImplementedError("write your pallas kernel here")


# speedup vs baseline: 1.1703x; 1.1703x over previous
import functools

import jax
import jax.numpy as jnp
from jax import lax
from jax.experimental import pallas as pl
from jax.experimental.pallas import tpu as pltpu
from jax.experimental.pallas import tpu_sc as plsc

SIZE = 128
NSCALE = 4
D = NSCALE * SIZE

NC = 2
NS = 16
NW = NC * NS

CHUNK = 128


def _sc_gather(n_pad, n_chunks_per_worker):
    rows_per_worker = n_chunks_per_worker * CHUNK
    mesh = plsc.VectorSubcoreMesh(
        core_axis_name="c", subcore_axis_name="s",
        num_cores=NC, num_subcores=NS)

    @functools.partial(
        pl.kernel,
        out_type=[jax.ShapeDtypeStruct((n_pad, SIZE), jnp.float32)
                  for _ in range(NSCALE)],
        mesh=mesh,
        scratch_types=[
            pltpu.VMEM((n_chunks_per_worker, CHUNK), jnp.int32),
            pltpu.VMEM((CHUNK, SIZE), jnp.float32),
            pltpu.SemaphoreType.DMA,
        ],
    )
    def sc_kernel(x0, x1, x2, x3, s0, s1, s2, s3,
                  o0, o1, o2, o3, idx_v, buf, sem):
        wid = lax.axis_index("s") * NC + lax.axis_index("c")
        base = pl.multiple_of(wid * rows_per_worker, 8)

        for x, s, o in ((x0, s0, o0), (x1, s1, o1), (x2, s2, o2), (x3, s3, o3)):
            pltpu.sync_copy(s.at[wid], idx_v)

            def body(j, _, x=x, o=o):
                row = pl.multiple_of(base + j * CHUNK, 8)
                pltpu.async_copy(x.at[idx_v.at[j]], buf, sem).wait()
                pltpu.sync_copy(buf, o.at[pl.ds(row, CHUNK)])
                return ()

            lax.fori_loop(0, n_chunks_per_worker, body, (), unroll=False)

    return sc_kernel


def _tc_body(g0, g1, g2, g3, w, b, gamma, beta, o):
    dn = (((1,), (1,)), ((), ()))
    acc = lax.dot_general(g0[...], w[:, 0 * SIZE:1 * SIZE], dn,
                          preferred_element_type=jnp.float32)
    acc += lax.dot_general(g1[...], w[:, 1 * SIZE:2 * SIZE], dn,
                           preferred_element_type=jnp.float32)
    acc += lax.dot_general(g2[...], w[:, 2 * SIZE:3 * SIZE], dn,
                           preferred_element_type=jnp.float32)
    acc += lax.dot_general(g3[...], w[:, 3 * SIZE:4 * SIZE], dn,
                           preferred_element_type=jnp.float32)
    y = jnp.maximum(acc + b[...], 0.0)
    mean = jnp.mean(y, axis=1, keepdims=True)
    yc = y - mean
    var = jnp.mean(yc * yc, axis=1, keepdims=True)
    o[...] = yc * lax.rsqrt(var + 1e-6) * gamma[...] + beta[...]


def kernel(x_friend, x_follow, x_group, x_event,
           src_friend, src_follow, src_group, src_event,
           W, b, gamma, beta):
    n = x_friend.shape[0]
    rows_q = NW * CHUNK
    n_chunks_per_worker = -(-n // rows_q)
    n_pad = n_chunks_per_worker * rows_q
    pad = n_pad - n

    def prep(s):
        s = jnp.concatenate([s, jnp.zeros((pad,), jnp.int32)])
        return s.reshape(NW, n_chunks_per_worker, CHUNK)

    srcs = [prep(s) for s in (src_friend, src_follow, src_group, src_event)]
    xs = [x_friend, x_follow, x_group, x_event]

    g0, g1, g2, g3 = _sc_gather(n_pad, n_chunks_per_worker)(*xs, *srcs)

    bn = 512
    grid = -(-n // bn)
    out = pl.pallas_call(
        _tc_body,
        grid=(grid,),
        in_specs=[
            pl.BlockSpec((bn, SIZE), lambda i: (i, 0)),
            pl.BlockSpec((bn, SIZE), lambda i: (i, 0)),
            pl.BlockSpec((bn, SIZE), lambda i: (i, 0)),
            pl.BlockSpec((bn, SIZE), lambda i: (i, 0)),
            pl.BlockSpec((D, D), lambda i: (0, 0)),
            pl.BlockSpec((1, D), lambda i: (0, 0)),
            pl.BlockSpec((1, D), lambda i: (0, 0)),
            pl.BlockSpec((1, D), lambda i: (0, 0)),
        ],
        out_specs=pl.BlockSpec((bn, D), lambda i: (i, 0)),
        out_shape=jax.ShapeDtypeStruct((n, D), jnp.float32),
    )(g0, g1, g2, g3, W,
      b.reshape(1, D), gamma.reshape(1, D), beta.reshape(1, D))
    return out


# speedup vs baseline: 2.1762x; 1.8595x over previous
import functools

import jax
import jax.numpy as jnp
from jax import lax
from jax.experimental import pallas as pl
from jax.experimental.pallas import tpu as pltpu
from jax.experimental.pallas import tpu_sc as plsc

SIZE = 128
NSCALE = 4
D = NSCALE * SIZE

NC = 2
NS = 16
NW = NC * NS

CHUNK = 112
RING = 4


def _sc_gather(n_pad, nch):
    rows_per_worker = nch * CHUNK
    mesh = plsc.VectorSubcoreMesh(
        core_axis_name="c", subcore_axis_name="s",
        num_cores=NC, num_subcores=NS)

    @functools.partial(
        pl.kernel,
        out_type=[jax.ShapeDtypeStruct((n_pad, SIZE), jnp.float32)
                  for _ in range(NSCALE)],
        mesh=mesh,
        scratch_types=(
            [pltpu.VMEM((nch, CHUNK), jnp.int32) for _ in range(NSCALE)]
            + [pltpu.VMEM((CHUNK, SIZE), jnp.float32) for _ in range(RING)]
            + [pltpu.SemaphoreType.DMA for _ in range(NSCALE)]
            + [pltpu.SemaphoreType.DMA for _ in range(RING)]
            + [pltpu.SemaphoreType.DMA for _ in range(RING)]
        ),
    )
    def sc_kernel(x0, x1, x2, x3, s0, s1, s2, s3, o0, o1, o2, o3,
                  i0, i1, i2, i3, b0, b1, b2, b3,
                  si0, si1, si2, si3, sg0, sg1, sg2, sg3,
                  ss0, ss1, ss2, ss3):
        xs = (x0, x1, x2, x3)
        ss = (s0, s1, s2, s3)
        os_ = (o0, o1, o2, o3)
        idxs = (i0, i1, i2, i3)
        bufs = (b0, b1, b2, b3)
        isems = (si0, si1, si2, si3)
        gsems = (sg0, sg1, sg2, sg3)
        ssems = (ss0, ss1, ss2, ss3)

        wid = lax.axis_index("s") * NC + lax.axis_index("c")
        base = pl.multiple_of(wid * rows_per_worker, 8)

        for t in range(NSCALE):
            pltpu.make_async_copy(ss[t].at[wid], idxs[t], isems[t]).start()

        for t in range(NSCALE):
            x, o, idx = xs[t], os_[t], idxs[t]

            def gather(c, r, x=x, idx=idx):
                return pltpu.make_async_copy(x.at[idx.at[c]], bufs[r],
                                             gsems[r])

            def store(c, r, o=o):
                row = pl.multiple_of(base + c * CHUNK, 8)
                return pltpu.make_async_copy(bufs[r], o.at[pl.ds(row, CHUNK)],
                                             ssems[r])

            pltpu.make_async_copy(ss[t].at[wid], idx, isems[t]).wait()
            for r in range(RING):
                gather(r, r).start()

            def body(j0, _, gather=gather, store=store):
                c0 = j0 * RING
                for r in range(RING):
                    gather(c0 + r, r).wait()
                    store(c0 + r, r).start()
                for r in range(RING):
                    store(c0 + r, r).wait()
                    gather(c0 + r + RING, r).start()
                return ()

            lax.fori_loop(0, nch // RING - 1, body, (), unroll=False)

            c0 = nch - RING
            for r in range(RING):
                gather(c0 + r, r).wait()
                store(c0 + r, r).start()
            for r in range(RING):
                store(c0 + r, r).wait()

    return sc_kernel


def _tc_body(g0, g1, g2, g3, w, b, gamma, beta, o):
    dn = (((1,), (1,)), ((), ()))
    acc = lax.dot_general(g0[...], w[:, 0 * SIZE:1 * SIZE], dn,
                          preferred_element_type=jnp.float32)
    acc += lax.dot_general(g1[...], w[:, 1 * SIZE:2 * SIZE], dn,
                           preferred_element_type=jnp.float32)
    acc += lax.dot_general(g2[...], w[:, 2 * SIZE:3 * SIZE], dn,
                           preferred_element_type=jnp.float32)
    acc += lax.dot_general(g3[...], w[:, 3 * SIZE:4 * SIZE], dn,
                           preferred_element_type=jnp.float32)
    y = jnp.maximum(acc + b[...], 0.0)
    mean = jnp.mean(y, axis=1, keepdims=True)
    yc = y - mean
    var = jnp.mean(yc * yc, axis=1, keepdims=True)
    o[...] = yc * lax.rsqrt(var + 1e-6) * gamma[...] + beta[...]


def kernel(x_friend, x_follow, x_group, x_event,
           src_friend, src_follow, src_group, src_event,
           W, b, gamma, beta):
    n = x_friend.shape[0]
    rows_q = NW * CHUNK
    nch = -(-n // rows_q)
    if nch % RING:
        nch += RING - nch % RING
    n_pad = nch * rows_q
    pad = n_pad - n

    def prep(s):
        s = jnp.concatenate([s, jnp.zeros((pad,), jnp.int32)])
        return s.reshape(NW, nch, CHUNK)

    srcs = [prep(s) for s in (src_friend, src_follow, src_group, src_event)]
    xs = [x_friend, x_follow, x_group, x_event]

    g0, g1, g2, g3 = _sc_gather(n_pad, nch)(*xs, *srcs)

    bn = 512
    grid = -(-n // bn)
    out = pl.pallas_call(
        _tc_body,
        grid=(grid,),
        in_specs=[
            pl.BlockSpec((bn, SIZE), lambda i: (i, 0)),
            pl.BlockSpec((bn, SIZE), lambda i: (i, 0)),
            pl.BlockSpec((bn, SIZE), lambda i: (i, 0)),
            pl.BlockSpec((bn, SIZE), lambda i: (i, 0)),
            pl.BlockSpec((D, D), lambda i: (0, 0)),
            pl.BlockSpec((1, D), lambda i: (0, 0)),
            pl.BlockSpec((1, D), lambda i: (0, 0)),
            pl.BlockSpec((1, D), lambda i: (0, 0)),
        ],
        out_specs=pl.BlockSpec((bn, D), lambda i: (i, 0)),
        out_shape=jax.ShapeDtypeStruct((n, D), jnp.float32),
    )(g0, g1, g2, g3, W,
      b.reshape(1, D), gamma.reshape(1, D), beta.reshape(1, D))
    return out


# speedup vs baseline: 2.1973x; 1.0097x over previous
import functools

import jax
import jax.numpy as jnp
from jax import lax
from jax.experimental import pallas as pl
from jax.experimental.pallas import tpu as pltpu
from jax.experimental.pallas import tpu_sc as plsc

SIZE = 128
NSCALE = 4
D = NSCALE * SIZE

NC = 2
NS = 16
NW = NC * NS

CHUNK = 112
RING = 7


def _sc_gather(n_pad, nch):
    rows_per_worker = nch * CHUNK
    mesh = plsc.VectorSubcoreMesh(
        core_axis_name="c", subcore_axis_name="s",
        num_cores=NC, num_subcores=NS)

    @functools.partial(
        pl.kernel,
        out_type=[jax.ShapeDtypeStruct((n_pad, SIZE), jnp.float32)
                  for _ in range(NSCALE)],
        mesh=mesh,
        scratch_types=(
            [pltpu.VMEM((nch, CHUNK), jnp.int32) for _ in range(NSCALE)]
            + [pltpu.VMEM((CHUNK, SIZE), jnp.float32) for _ in range(RING)]
            + [pltpu.SemaphoreType.DMA for _ in range(NSCALE)]
            + [pltpu.SemaphoreType.DMA for _ in range(RING)]
            + [pltpu.SemaphoreType.DMA for _ in range(RING)]
        ),
    )
    def sc_kernel(*refs):
        xs = refs[0:NSCALE]
        ss = refs[NSCALE:2 * NSCALE]
        os_ = refs[2 * NSCALE:3 * NSCALE]
        rest = refs[3 * NSCALE:]
        idxs = rest[0:NSCALE]
        bufs = rest[NSCALE:NSCALE + RING]
        isems = rest[NSCALE + RING:2 * NSCALE + RING]
        gsems = rest[2 * NSCALE + RING:2 * NSCALE + 2 * RING]
        ssems = rest[2 * NSCALE + 2 * RING:2 * NSCALE + 3 * RING]

        wid = lax.axis_index("s") * NC + lax.axis_index("c")
        base = pl.multiple_of(wid * rows_per_worker, 8)

        for t in range(NSCALE):
            pltpu.make_async_copy(ss[t].at[wid], idxs[t], isems[t]).start()

        for t in range(NSCALE):
            x, o, idx = xs[t], os_[t], idxs[t]

            def gather(c, r, x=x, idx=idx):
                return pltpu.make_async_copy(x.at[idx.at[c]], bufs[r],
                                             gsems[r])

            def store(c, r, o=o):
                row = pl.multiple_of(base + c * CHUNK, 8)
                return pltpu.make_async_copy(bufs[r], o.at[pl.ds(row, CHUNK)],
                                             ssems[r])

            pltpu.make_async_copy(ss[t].at[wid], idx, isems[t]).wait()
            for r in range(RING):
                gather(r, r).start()

            def body(j0, _, gather=gather, store=store):
                c0 = j0 * RING
                for r in range(RING):
                    gather(c0 + r, r).wait()
                    store(c0 + r, r).start()
                for r in range(RING):
                    store(c0 + r, r).wait()
                    gather(c0 + r + RING, r).start()
                return ()

            lax.fori_loop(0, nch // RING - 1, body, (), unroll=False)

            c0 = nch - RING
            for r in range(RING):
                gather(c0 + r, r).wait()
                store(c0 + r, r).start()
            for r in range(RING):
                store(c0 + r, r).wait()

    return sc_kernel


def _tc_body(g0, g1, g2, g3, w, b, gamma, beta, o):
    dn = (((1,), (1,)), ((), ()))
    acc = lax.dot_general(g0[...], w[:, 0 * SIZE:1 * SIZE], dn,
                          preferred_element_type=jnp.float32)
    acc += lax.dot_general(g1[...], w[:, 1 * SIZE:2 * SIZE], dn,
                           preferred_element_type=jnp.float32)
    acc += lax.dot_general(g2[...], w[:, 2 * SIZE:3 * SIZE], dn,
                           preferred_element_type=jnp.float32)
    acc += lax.dot_general(g3[...], w[:, 3 * SIZE:4 * SIZE], dn,
                           preferred_element_type=jnp.float32)
    y = jnp.maximum(acc + b[...], 0.0)
    mean = jnp.mean(y, axis=1, keepdims=True)
    yc = y - mean
    var = jnp.mean(yc * yc, axis=1, keepdims=True)
    o[...] = yc * lax.rsqrt(var + 1e-6) * gamma[...] + beta[...]


def kernel(x_friend, x_follow, x_group, x_event,
           src_friend, src_follow, src_group, src_event,
           W, b, gamma, beta):
    n = x_friend.shape[0]
    rows_q = NW * CHUNK
    nch = -(-n // rows_q)
    if nch % RING:
        nch += RING - nch % RING
    n_pad = nch * rows_q
    pad = n_pad - n

    def prep(s):
        s = jnp.concatenate([s, jnp.zeros((pad,), jnp.int32)])
        return s.reshape(NW, nch, CHUNK)

    srcs = [prep(s) for s in (src_friend, src_follow, src_group, src_event)]
    xs = [x_friend, x_follow, x_group, x_event]

    g0, g1, g2, g3 = _sc_gather(n_pad, nch)(*xs, *srcs)

    bn = 512
    grid = -(-n // bn)
    out = pl.pallas_call(
        _tc_body,
        grid=(grid,),
        in_specs=[
            pl.BlockSpec((bn, SIZE), lambda i: (i, 0)),
            pl.BlockSpec((bn, SIZE), lambda i: (i, 0)),
            pl.BlockSpec((bn, SIZE), lambda i: (i, 0)),
            pl.BlockSpec((bn, SIZE), lambda i: (i, 0)),
            pl.BlockSpec((D, D), lambda i: (0, 0)),
            pl.BlockSpec((1, D), lambda i: (0, 0)),
            pl.BlockSpec((1, D), lambda i: (0, 0)),
            pl.BlockSpec((1, D), lambda i: (0, 0)),
        ],
        out_specs=pl.BlockSpec((bn, D), lambda i: (i, 0)),
        out_shape=jax.ShapeDtypeStruct((n, D), jnp.float32),
    )(g0, g1, g2, g3, W,
      b.reshape(1, D), gamma.reshape(1, D), beta.reshape(1, D))
    return out


# speedup vs baseline: 2.6884x; 1.2235x over previous
import functools

import jax
import jax.numpy as jnp
from jax import lax
from jax.experimental import pallas as pl
from jax.experimental.pallas import tpu as pltpu
from jax.experimental.pallas import tpu_sc as plsc

SIZE = 128
NSCALE = 4
D = NSCALE * SIZE

NC = 2
NS = 16
NW = NC * NS

CHUNK = 112
RING = 7


def _sc_gather(n_pad, nch):
    rows_per_worker = nch * CHUNK
    mesh = plsc.VectorSubcoreMesh(
        core_axis_name="c", subcore_axis_name="s",
        num_cores=NC, num_subcores=NS)

    @functools.partial(
        pl.kernel,
        out_type=[jax.ShapeDtypeStruct((n_pad, SIZE), jnp.float32)
                  for _ in range(NSCALE)],
        mesh=mesh,
        scratch_types=(
            [pltpu.VMEM((nch, CHUNK), jnp.int32) for _ in range(NSCALE)]
            + [pltpu.VMEM((CHUNK, SIZE), jnp.float32) for _ in range(RING)]
            + [pltpu.SemaphoreType.DMA for _ in range(NSCALE)]
            + [pltpu.SemaphoreType.DMA for _ in range(RING)]
            + [pltpu.SemaphoreType.DMA for _ in range(RING)]
        ),
    )
    def sc_kernel(*refs):
        xs = refs[0:NSCALE]
        ss = refs[NSCALE:2 * NSCALE]
        os_ = refs[2 * NSCALE:3 * NSCALE]
        rest = refs[3 * NSCALE:]
        idxs = rest[0:NSCALE]
        bufs = rest[NSCALE:NSCALE + RING]
        isems = rest[NSCALE + RING:2 * NSCALE + RING]
        gsems = rest[2 * NSCALE + RING:2 * NSCALE + 2 * RING]
        ssems = rest[2 * NSCALE + 2 * RING:2 * NSCALE + 3 * RING]

        wid = lax.axis_index("s") * NC + lax.axis_index("c")
        base = pl.multiple_of(wid * rows_per_worker, 8)

        for t in range(NSCALE):
            pltpu.make_async_copy(ss[t].at[wid], idxs[t], isems[t]).start()

        for t in range(NSCALE):
            x, o, idx = xs[t], os_[t], idxs[t]

            def gather(c, r, x=x, idx=idx):
                return pltpu.make_async_copy(x.at[idx.at[c]], bufs[r],
                                             gsems[r])

            def store(c, r, o=o):
                row = pl.multiple_of(base + c * CHUNK, 8)
                return pltpu.make_async_copy(bufs[r], o.at[pl.ds(row, CHUNK)],
                                             ssems[r])

            pltpu.make_async_copy(ss[t].at[wid], idx, isems[t]).wait()
            for r in range(RING):
                gather(r, r).start()

            def body(j0, _, gather=gather, store=store):
                c0 = j0 * RING
                for r in range(RING):
                    gather(c0 + r, r).wait()
                    store(c0 + r, r).start()
                for r in range(RING):
                    store(c0 + r, r).wait()
                    gather(c0 + r + RING, r).start()
                return ()

            lax.fori_loop(0, nch // RING - 1, body, (), unroll=False)

            c0 = nch - RING
            for r in range(RING):
                gather(c0 + r, r).wait()
                store(c0 + r, r).start()
            for r in range(RING):
                store(c0 + r, r).wait()

    return sc_kernel


def _tc_compute(g0, g1, g2, g3, w, b, gamma, beta, o):
    dn = (((1,), (1,)), ((), ()))
    acc = lax.dot_general(g0[...], w[:, 0 * SIZE:1 * SIZE], dn,
                          preferred_element_type=jnp.float32)
    acc += lax.dot_general(g1[...], w[:, 1 * SIZE:2 * SIZE], dn,
                           preferred_element_type=jnp.float32)
    acc += lax.dot_general(g2[...], w[:, 2 * SIZE:3 * SIZE], dn,
                           preferred_element_type=jnp.float32)
    acc += lax.dot_general(g3[...], w[:, 3 * SIZE:4 * SIZE], dn,
                           preferred_element_type=jnp.float32)
    y = jnp.maximum(acc + b[...], 0.0)
    mean = jnp.mean(y, axis=1, keepdims=True)
    yc = y - mean
    var = jnp.mean(yc * yc, axis=1, keepdims=True)
    o[...] = yc * lax.rsqrt(var + 1e-6) * gamma[...] + beta[...]


SLABS = 2
BN = 512


def kernel(x_friend, x_follow, x_group, x_event,
           src_friend, src_follow, src_group, src_event,
           W, b, gamma, beta):
    n = x_friend.shape[0]
    rows_q = NW * CHUNK * SLABS
    nch = -(-n // rows_q)
    if nch % RING:
        nch += RING - nch % RING
    n_pad = nch * rows_q
    pad = n_pad - n
    rows_slab = n_pad // SLABS
    blocks_s = rows_slab // BN

    def prep(s):
        s = jnp.concatenate([s, jnp.zeros((pad,), jnp.int32)])
        return s.reshape(SLABS, NW, nch, CHUNK)

    srcs = [prep(s) for s in (src_friend, src_follow, src_group, src_event)]
    xs = [x_friend, x_follow, x_group, x_event]
    b2, gamma2, beta2 = b.reshape(1, D), gamma.reshape(1, D), beta.reshape(1, D)

    sc = _sc_gather(rows_slab, nch)
    gs = [sc(*xs, *(s[k] for s in srcs)) for k in range(SLABS)]

    g_spec = pl.BlockSpec((BN, SIZE), lambda i: (i, 0))
    common_specs = [
        g_spec, g_spec, g_spec, g_spec,
        pl.BlockSpec((D, D), lambda i: (0, 0)),
        pl.BlockSpec((1, D), lambda i: (0, 0)),
        pl.BlockSpec((1, D), lambda i: (0, 0)),
        pl.BlockSpec((1, D), lambda i: (0, 0)),
    ]

    out = None
    for k in range(SLABS):
        def out_map(i, k=k):
            return (k * blocks_s + i, 0)
        if k == 0:
            out = pl.pallas_call(
                _tc_compute,
                grid=(blocks_s,),
                in_specs=common_specs,
                out_specs=pl.BlockSpec((BN, D), out_map),
                out_shape=jax.ShapeDtypeStruct((n, D), jnp.float32),
            )(*gs[k], W, b2, gamma2, beta2)
        else:
            def body(g0, g1, g2, g3, w, bb, gam, bet, prev, o):
                _tc_compute(g0, g1, g2, g3, w, bb, gam, bet, o)
            out = pl.pallas_call(
                body,
                grid=(blocks_s,),
                in_specs=common_specs
                + [pl.BlockSpec(memory_space=pltpu.MemorySpace.HBM)],
                out_specs=pl.BlockSpec((BN, D), out_map),
                out_shape=jax.ShapeDtypeStruct((n, D), jnp.float32),
                input_output_aliases={8: 0},
            )(*gs[k], W, b2, gamma2, beta2, out)
    return out


# speedup vs baseline: 2.7919x; 1.0385x over previous
import functools

import jax
import jax.numpy as jnp
from jax import lax
from jax.experimental import pallas as pl
from jax.experimental.pallas import tpu as pltpu
from jax.experimental.pallas import tpu_sc as plsc

SIZE = 128
NSCALE = 4
D = NSCALE * SIZE

NC = 2
NS = 16
NW = NC * NS

CHUNK = 112
RING = 7


def _sc_gather(n_pad, nch):
    rows_per_worker = nch * CHUNK
    mesh = plsc.VectorSubcoreMesh(
        core_axis_name="c", subcore_axis_name="s",
        num_cores=NC, num_subcores=NS)

    @functools.partial(
        pl.kernel,
        out_type=[jax.ShapeDtypeStruct((n_pad, SIZE), jnp.float32)
                  for _ in range(NSCALE)],
        mesh=mesh,
        scratch_types=(
            [pltpu.VMEM((nch, CHUNK), jnp.int32) for _ in range(NSCALE)]
            + [pltpu.VMEM((CHUNK, SIZE), jnp.float32) for _ in range(RING)]
            + [pltpu.SemaphoreType.DMA for _ in range(NSCALE)]
            + [pltpu.SemaphoreType.DMA for _ in range(RING)]
            + [pltpu.SemaphoreType.DMA for _ in range(RING)]
        ),
    )
    def sc_kernel(*refs):
        xs = refs[0:NSCALE]
        ss = refs[NSCALE:2 * NSCALE]
        os_ = refs[2 * NSCALE:3 * NSCALE]
        rest = refs[3 * NSCALE:]
        idxs = rest[0:NSCALE]
        bufs = rest[NSCALE:NSCALE + RING]
        isems = rest[NSCALE + RING:2 * NSCALE + RING]
        gsems = rest[2 * NSCALE + RING:2 * NSCALE + 2 * RING]
        ssems = rest[2 * NSCALE + 2 * RING:2 * NSCALE + 3 * RING]

        wid = lax.axis_index("s") * NC + lax.axis_index("c")
        base = pl.multiple_of(wid * rows_per_worker, 8)

        for t in range(NSCALE):
            pltpu.make_async_copy(ss[t].at[wid], idxs[t], isems[t]).start()

        for t in range(NSCALE):
            x, o, idx = xs[t], os_[t], idxs[t]

            def gather(c, r, x=x, idx=idx):
                return pltpu.make_async_copy(x.at[idx.at[c]], bufs[r],
                                             gsems[r])

            def store(c, r, o=o):
                row = pl.multiple_of(base + c * CHUNK, 8)
                return pltpu.make_async_copy(bufs[r], o.at[pl.ds(row, CHUNK)],
                                             ssems[r])

            pltpu.make_async_copy(ss[t].at[wid], idx, isems[t]).wait()
            for r in range(RING):
                gather(r, r).start()

            def body(j0, _, gather=gather, store=store):
                c0 = j0 * RING
                for r in range(RING):
                    gather(c0 + r, r).wait()
                    store(c0 + r, r).start()
                for r in range(RING):
                    store(c0 + r, r).wait()
                    gather(c0 + r + RING, r).start()
                return ()

            lax.fori_loop(0, nch // RING - 1, body, (), unroll=False)

            c0 = nch - RING
            for r in range(RING):
                gather(c0 + r, r).wait()
                store(c0 + r, r).start()
            for r in range(RING):
                store(c0 + r, r).wait()

    return sc_kernel


def _tc_compute(g0, g1, g2, g3, w, b, gamma, beta, o):
    dn = (((1,), (1,)), ((), ()))
    acc = lax.dot_general(g0[...], w[:, 0 * SIZE:1 * SIZE], dn,
                          preferred_element_type=jnp.float32)
    acc += lax.dot_general(g1[...], w[:, 1 * SIZE:2 * SIZE], dn,
                           preferred_element_type=jnp.float32)
    acc += lax.dot_general(g2[...], w[:, 2 * SIZE:3 * SIZE], dn,
                           preferred_element_type=jnp.float32)
    acc += lax.dot_general(g3[...], w[:, 3 * SIZE:4 * SIZE], dn,
                           preferred_element_type=jnp.float32)
    y = jnp.maximum(acc + b[...], 0.0)
    mean = jnp.mean(y, axis=1, keepdims=True)
    yc = y - mean
    var = jnp.mean(yc * yc, axis=1, keepdims=True)
    o[...] = yc * lax.rsqrt(var + 1e-6) * gamma[...] + beta[...]


SLABS = 4
BN = 512


def kernel(x_friend, x_follow, x_group, x_event,
           src_friend, src_follow, src_group, src_event,
           W, b, gamma, beta):
    n = x_friend.shape[0]
    rows_q = NW * CHUNK * SLABS
    nch = -(-n // rows_q)
    if nch % RING:
        nch += RING - nch % RING
    n_pad = nch * rows_q
    pad = n_pad - n
    rows_slab = n_pad // SLABS
    blocks_s = rows_slab // BN

    def prep(s):
        s = jnp.concatenate([s, jnp.zeros((pad,), jnp.int32)])
        return s.reshape(SLABS, NW, nch, CHUNK)

    srcs = [prep(s) for s in (src_friend, src_follow, src_group, src_event)]
    xs = [x_friend, x_follow, x_group, x_event]
    b2, gamma2, beta2 = b.reshape(1, D), gamma.reshape(1, D), beta.reshape(1, D)

    sc = _sc_gather(rows_slab, nch)
    gs = [sc(*xs, *(s[k] for s in srcs)) for k in range(SLABS)]

    g_spec = pl.BlockSpec((BN, SIZE), lambda i: (i, 0))
    common_specs = [
        g_spec, g_spec, g_spec, g_spec,
        pl.BlockSpec((D, D), lambda i: (0, 0)),
        pl.BlockSpec((1, D), lambda i: (0, 0)),
        pl.BlockSpec((1, D), lambda i: (0, 0)),
        pl.BlockSpec((1, D), lambda i: (0, 0)),
    ]

    out = None
    for k in range(SLABS):
        def out_map(i, k=k):
            return (k * blocks_s + i, 0)
        if k == 0:
            out = pl.pallas_call(
                _tc_compute,
                grid=(blocks_s,),
                in_specs=common_specs,
                out_specs=pl.BlockSpec((BN, D), out_map),
                out_shape=jax.ShapeDtypeStruct((n, D), jnp.float32),
            )(*gs[k], W, b2, gamma2, beta2)
        else:
            def body(g0, g1, g2, g3, w, bb, gam, bet, prev, o):
                _tc_compute(g0, g1, g2, g3, w, bb, gam, bet, o)
            out = pl.pallas_call(
                body,
                grid=(blocks_s,),
                in_specs=common_specs
                + [pl.BlockSpec(memory_space=pltpu.MemorySpace.HBM)],
                out_specs=pl.BlockSpec((BN, D), out_map),
                out_shape=jax.ShapeDtypeStruct((n, D), jnp.float32),
                input_output_aliases={8: 0},
            )(*gs[k], W, b2, gamma2, beta2, out)
    return out


# speedup vs baseline: 2.8880x; 1.0344x over previous
import functools

import jax
import jax.numpy as jnp
from jax import lax
from jax.experimental import pallas as pl
from jax.experimental.pallas import tpu as pltpu
from jax.experimental.pallas import tpu_sc as plsc

SIZE = 128
NSCALE = 4
D = NSCALE * SIZE

NC = 2
NS = 16
NW = NC * NS

CHUNK = 112
RING = 7


def _sc_gather(n_pad, nch):
    rows_per_worker = nch * CHUNK
    mesh = plsc.VectorSubcoreMesh(
        core_axis_name="c", subcore_axis_name="s",
        num_cores=NC, num_subcores=NS)

    @functools.partial(
        pl.kernel,
        out_type=jax.ShapeDtypeStruct((n_pad, D), jnp.float32),
        mesh=mesh,
        scratch_types=(
            [pltpu.VMEM((nch, CHUNK), jnp.int32) for _ in range(NSCALE)]
            + [pltpu.VMEM((CHUNK, SIZE), jnp.float32) for _ in range(RING)]
            + [pltpu.SemaphoreType.DMA for _ in range(NSCALE)]
            + [pltpu.SemaphoreType.DMA for _ in range(RING)]
            + [pltpu.SemaphoreType.DMA for _ in range(RING)]
        ),
    )
    def sc_kernel(*refs):
        xs = refs[0:NSCALE]
        ss = refs[NSCALE:2 * NSCALE]
        o = refs[2 * NSCALE]
        rest = refs[2 * NSCALE + 1:]
        idxs = rest[0:NSCALE]
        bufs = rest[NSCALE:NSCALE + RING]
        isems = rest[NSCALE + RING:2 * NSCALE + RING]
        gsems = rest[2 * NSCALE + RING:2 * NSCALE + 2 * RING]
        ssems = rest[2 * NSCALE + 2 * RING:2 * NSCALE + 3 * RING]

        wid = lax.axis_index("s") * NC + lax.axis_index("c")
        base = pl.multiple_of(wid * rows_per_worker, 8)

        for t in range(NSCALE):
            pltpu.make_async_copy(ss[t].at[wid], idxs[t], isems[t]).start()

        for t in range(NSCALE):
            x, idx = xs[t], idxs[t]

            def gather(c, r, x=x, idx=idx):
                return pltpu.make_async_copy(x.at[idx.at[c]], bufs[r],
                                             gsems[r])

            def store(c, r, t=t):
                row = pl.multiple_of(base + c * CHUNK, 8)
                return pltpu.make_async_copy(
                    bufs[r],
                    o.at[pl.ds(row, CHUNK), pl.ds(t * SIZE, SIZE)],
                    ssems[r])

            pltpu.make_async_copy(ss[t].at[wid], idx, isems[t]).wait()
            for r in range(RING):
                gather(r, r).start()

            def body(j0, _, gather=gather, store=store):
                c0 = j0 * RING
                for r in range(RING):
                    gather(c0 + r, r).wait()
                    store(c0 + r, r).start()
                for r in range(RING):
                    store(c0 + r, r).wait()
                    gather(c0 + r + RING, r).start()
                return ()

            lax.fori_loop(0, nch // RING - 1, body, (), unroll=False)

            c0 = nch - RING
            for r in range(RING):
                gather(c0 + r, r).wait()
                store(c0 + r, r).start()
            for r in range(RING):
                store(c0 + r, r).wait()

    return sc_kernel


def _tc_compute(g, w, b, gamma, beta, o):
    dn = (((1,), (1,)), ((), ()))
    acc = lax.dot_general(g[...], w[...], dn,
                          preferred_element_type=jnp.float32)
    y = jnp.maximum(acc + b[...], 0.0)
    mean = jnp.mean(y, axis=1, keepdims=True)
    yc = y - mean
    var = jnp.mean(yc * yc, axis=1, keepdims=True)
    o[...] = yc * lax.rsqrt(var + 1e-6) * gamma[...] + beta[...]


SLABS = 4
BN = 512


def kernel(x_friend, x_follow, x_group, x_event,
           src_friend, src_follow, src_group, src_event,
           W, b, gamma, beta):
    n = x_friend.shape[0]
    rows_q = NW * CHUNK * SLABS
    nch = -(-n // rows_q)
    if nch % RING:
        nch += RING - nch % RING
    n_pad = nch * rows_q
    pad = n_pad - n
    rows_slab = n_pad // SLABS
    blocks_s = rows_slab // BN

    def prep(s):
        s = jnp.concatenate([s, jnp.zeros((pad,), jnp.int32)])
        return s.reshape(SLABS, NW, nch, CHUNK)

    srcs = [prep(s) for s in (src_friend, src_follow, src_group, src_event)]
    xs = [x_friend, x_follow, x_group, x_event]
    b2, gamma2, beta2 = b.reshape(1, D), gamma.reshape(1, D), beta.reshape(1, D)

    sc = _sc_gather(rows_slab, nch)
    gs = [sc(*xs, *(s[k] for s in srcs)) for k in range(SLABS)]

    common_specs = [
        pl.BlockSpec((BN, D), lambda i: (i, 0)),
        pl.BlockSpec((D, D), lambda i: (0, 0)),
        pl.BlockSpec((1, D), lambda i: (0, 0)),
        pl.BlockSpec((1, D), lambda i: (0, 0)),
        pl.BlockSpec((1, D), lambda i: (0, 0)),
    ]

    out = None
    for k in range(SLABS):
        def out_map(i, k=k):
            return (k * blocks_s + i, 0)
        if k == 0:
            out = pl.pallas_call(
                _tc_compute,
                grid=(blocks_s,),
                in_specs=common_specs,
                out_specs=pl.BlockSpec((BN, D), out_map),
                out_shape=jax.ShapeDtypeStruct((n, D), jnp.float32),
            )(gs[k], W, b2, gamma2, beta2)
        else:
            def body(g, w, bb, gam, bet, prev, o):
                _tc_compute(g, w, bb, gam, bet, o)
            out = pl.pallas_call(
                body,
                grid=(blocks_s,),
                in_specs=common_specs
                + [pl.BlockSpec(memory_space=pltpu.MemorySpace.HBM)],
                out_specs=pl.BlockSpec((BN, D), out_map),
                out_shape=jax.ShapeDtypeStruct((n, D), jnp.float32),
                input_output_aliases={5: 0},
            )(gs[k], W, b2, gamma2, beta2, out)
    return out


# speedup vs baseline: 3.0571x; 1.0585x over previous
import functools

import jax
import jax.numpy as jnp
from jax import lax
from jax.experimental import pallas as pl
from jax.experimental.pallas import tpu as pltpu
from jax.experimental.pallas import tpu_sc as plsc

SIZE = 128
NSCALE = 4
D = NSCALE * SIZE

NC = 2
NS = 16
NW = NC * NS

CHUNK = 112
RING = 7


def _sc_gather(n_pad, nch):
    rows_per_worker = nch * CHUNK
    mesh = plsc.VectorSubcoreMesh(
        core_axis_name="c", subcore_axis_name="s",
        num_cores=NC, num_subcores=NS)

    @functools.partial(
        pl.kernel,
        out_type=jax.ShapeDtypeStruct((n_pad, D), jnp.float32),
        mesh=mesh,
        scratch_types=(
            [pltpu.VMEM((nch, CHUNK), jnp.int32) for _ in range(NSCALE)]
            + [pltpu.VMEM((CHUNK, SIZE), jnp.float32) for _ in range(RING)]
            + [pltpu.SemaphoreType.DMA for _ in range(NSCALE)]
            + [pltpu.SemaphoreType.DMA for _ in range(RING)]
            + [pltpu.SemaphoreType.DMA for _ in range(RING)]
        ),
    )
    def sc_kernel(*refs):
        xs = refs[0:NSCALE]
        ss = refs[NSCALE:2 * NSCALE]
        o = refs[2 * NSCALE]
        rest = refs[2 * NSCALE + 1:]
        idxs = rest[0:NSCALE]
        bufs = rest[NSCALE:NSCALE + RING]
        isems = rest[NSCALE + RING:2 * NSCALE + RING]
        gsems = rest[2 * NSCALE + RING:2 * NSCALE + 2 * RING]
        ssems = rest[2 * NSCALE + 2 * RING:2 * NSCALE + 3 * RING]

        wid = lax.axis_index("s") * NC + lax.axis_index("c")
        base = pl.multiple_of(wid * rows_per_worker, 8)

        for t in range(NSCALE):
            pltpu.make_async_copy(ss[t].at[wid], idxs[t], isems[t]).start()

        for t in range(NSCALE):
            x, idx = xs[t], idxs[t]

            def gather(c, r, x=x, idx=idx):
                return pltpu.make_async_copy(x.at[idx.at[c]], bufs[r],
                                             gsems[r])

            def store(c, r, t=t):
                row = pl.multiple_of(base + c * CHUNK, 8)
                return pltpu.make_async_copy(
                    bufs[r],
                    o.at[pl.ds(row, CHUNK), pl.ds(t * SIZE, SIZE)],
                    ssems[r])

            pltpu.make_async_copy(ss[t].at[wid], idx, isems[t]).wait()
            for r in range(RING):
                gather(r, r).start()

            def body(j0, _, gather=gather, store=store):
                c0 = j0 * RING
                for r in range(RING):
                    gather(c0 + r, r).wait()
                    store(c0 + r, r).start()
                for r in range(RING):
                    store(c0 + r, r).wait()
                    gather(c0 + r + RING, r).start()
                return ()

            lax.fori_loop(0, nch // RING - 1, body, (), unroll=False)

            c0 = nch - RING
            for r in range(RING):
                gather(c0 + r, r).wait()
                store(c0 + r, r).start()
            for r in range(RING):
                store(c0 + r, r).wait()

    return sc_kernel


def _tc_compute(g, w, b, gamma, beta, o):
    dn = (((1,), (1,)), ((), ()))
    acc = lax.dot_general(g[...], w[...], dn,
                          preferred_element_type=jnp.float32)
    y = jnp.maximum(acc + b[...], 0.0)
    mean = jnp.mean(y, axis=1, keepdims=True)
    yc = y - mean
    var = jnp.mean(yc * yc, axis=1, keepdims=True)
    o[...] = yc * lax.rsqrt(var + 1e-6) * gamma[...] + beta[...]


SLABS = 4
BN = 1024


def kernel(x_friend, x_follow, x_group, x_event,
           src_friend, src_follow, src_group, src_event,
           W, b, gamma, beta):
    n = x_friend.shape[0]
    rows_q = NW * CHUNK * SLABS
    nch = -(-n // rows_q)
    if nch % RING:
        nch += RING - nch % RING
    n_pad = nch * rows_q
    pad = n_pad - n
    rows_slab = n_pad // SLABS
    blocks_s = rows_slab // BN

    def prep(s):
        s = jnp.concatenate([s, jnp.zeros((pad,), jnp.int32)])
        return s.reshape(SLABS, NW, nch, CHUNK)

    srcs = [prep(s) for s in (src_friend, src_follow, src_group, src_event)]
    xs = [x_friend, x_follow, x_group, x_event]
    b2, gamma2, beta2 = b.reshape(1, D), gamma.reshape(1, D), beta.reshape(1, D)

    sc = _sc_gather(rows_slab, nch)
    gs = [sc(*xs, *(s[k] for s in srcs)) for k in range(SLABS)]

    common_specs = [
        pl.BlockSpec((BN, D), lambda i: (i, 0)),
        pl.BlockSpec((D, D), lambda i: (0, 0)),
        pl.BlockSpec((1, D), lambda i: (0, 0)),
        pl.BlockSpec((1, D), lambda i: (0, 0)),
        pl.BlockSpec((1, D), lambda i: (0, 0)),
    ]

    out = None
    for k in range(SLABS):
        def out_map(i, k=k):
            return (k * blocks_s + i, 0)
        if k == 0:
            out = pl.pallas_call(
                _tc_compute,
                grid=(blocks_s,),
                in_specs=common_specs,
                out_specs=pl.BlockSpec((BN, D), out_map),
                out_shape=jax.ShapeDtypeStruct((n, D), jnp.float32),
            )(gs[k], W, b2, gamma2, beta2)
        else:
            def body(g, w, bb, gam, bet, prev, o):
                _tc_compute(g, w, bb, gam, bet, o)
            out = pl.pallas_call(
                body,
                grid=(blocks_s,),
                in_specs=common_specs
                + [pl.BlockSpec(memory_space=pltpu.MemorySpace.HBM)],
                out_specs=pl.BlockSpec((BN, D), out_map),
                out_shape=jax.ShapeDtypeStruct((n, D), jnp.float32),
                input_output_aliases={5: 0},
            )(gs[k], W, b2, gamma2, beta2, out)
    return out
